# baseline (speedup 1.0000x reference)
import jax
import jax.numpy as jnp
from jax import lax
from jax.experimental import pallas as pl
from jax.experimental.pallas import tpu as pltpu
from jax.experimental.pallas import tpu_sc as plsc

_N = 10000
_E = 320000
_D = 128
_G = 256
_C = 60

_NC = 2
_NS = 16
_NW = _NC * _NS
_EPW = _E // _NW
_CH = 128
_NFULL = _EPW // _CH
_REM = _EPW - _NFULL * _CH
_NB = 3
_NR = _NFULL // _NB

_CHS = 128
_NFULLS = _EPW // _CHS
_REMS = _EPW - _NFULLS * _CHS
_NB4 = 2
_NRS = _NFULLS // _NB4

_RA = 624
_RLAST_EXTRA = _N - 16 * _RA

_mesh = plsc.VectorSubcoreMesh(
    core_axis_name="c", subcore_axis_name="s", num_cores=_NC, num_subcores=_NS
)


def _copy_idx_regs(dst_ref, src_ref, base, n):
    for j in range(n // 16):
        dst_ref[pl.ds(j * 16, 16)] = src_ref[pl.ds(base + j * 16, 16)]


def _deg_body(dst_hbm, out_hbm, acc, didx_all, zval, ones, idx16, ones16,
              dc0, dc1, dc2, isem, ss0, ss1, ss2):
    cid = lax.axis_index("c")
    sid = lax.axis_index("s")
    wid = sid * _NC + cid
    base0 = wid * _EPW
    dcur = (dc0, dc1, dc2)
    ssem = (ss0, ss1, ss2)

    idesc = pltpu.async_copy(dst_hbm.at[pl.ds(base0, _EPW)], didx_all, isem)

    def fill_z(i, _):
        zval[pl.ds(i * 16, 16)] = jnp.zeros((16,), jnp.float32)
        return 0

    lax.fori_loop(0, 640 // 16, fill_z, 0)

    def fill_o(i, _):
        ones[pl.ds(i * 16, 16)] = jnp.ones((16,), jnp.float32)
        return 0

    lax.fori_loop(0, _CH // 16, fill_o, 0)
    ones16[pl.ds(0, 16)] = jnp.ones((16,), jnp.float32)

    @pl.when(sid < _NS - 1)
    def _():
        pltpu.sync_copy(zval.at[pl.ds(0, _RA)], acc.at[pl.ds(sid * _RA, _RA)])

    @pl.when(sid == _NS - 1)
    def _():
        pltpu.sync_copy(zval, acc.at[pl.ds((_NS - 1) * _RA, _RA + _RLAST_EXTRA)])

    idesc.wait()
    plsc.subcore_barrier()

    def rnd(r, _):
        for b in range(_NB):
            i = r * _NB + b

            @pl.when(r > 0)
            def _():
                pltpu.make_async_copy(ones, acc.at[dcur[b]], ssem[b]).wait()

            _copy_idx_regs(dcur[b], didx_all, i * _CH, _CH)
            pltpu.async_copy(ones, acc.at[dcur[b]], ssem[b], add=True)
        return 0

    lax.fori_loop(0, _NR, rnd, 0)
    for b in range(_NB):
        pltpu.make_async_copy(ones, acc.at[dcur[b]], ssem[b]).wait()

    idx16[pl.ds(0, _REM)] = didx_all[pl.ds(_NFULL * _CH, _REM)]
    pltpu.sync_copy(ones16, acc.at[idx16], add=True)

    plsc.subcore_barrier()

    @pl.when(sid == 0)
    def _():
        pltpu.sync_copy(acc, out_hbm.at[cid])


_deg_call = pl.kernel(
    _deg_body,
    out_type=jax.ShapeDtypeStruct((_NC, _N), jnp.float32),
    mesh=_mesh,
    scratch_types=[
        pltpu.VMEM_SHARED((_N,), jnp.float32),
        pltpu.VMEM((_EPW,), jnp.int32),
        pltpu.VMEM((640,), jnp.float32),
        pltpu.VMEM((_CH,), jnp.float32),
        pltpu.VMEM((_REM,), jnp.int32),
        pltpu.VMEM((_REM,), jnp.float32),
        pltpu.VMEM((_CH,), jnp.int32),
        pltpu.VMEM((_CH,), jnp.int32),
        pltpu.VMEM((_CH,), jnp.int32),
        pltpu.SemaphoreType.DMA,
        pltpu.SemaphoreType.DMA,
        pltpu.SemaphoreType.DMA,
        pltpu.SemaphoreType.DMA,
    ],
)


def _scat_body(u_hbm, src_hbm, dst_hbm, out_hbm, acc, sidx_all,
               r0, r1, dc0, dc1, rows16, sidx16, didx16,
               isem, gs0, gs1, ss0, ss1, ds0, ds1):
    cid = lax.axis_index("c")
    sid = lax.axis_index("s")
    wid = sid * _NC + cid
    base0 = wid * _EPW
    rows = (r0, r1)
    dcur = (dc0, dc1)
    gsem = (gs0, gs1)
    ssem = (ss0, ss1)
    dsem = (ds0, ds1)

    isd = pltpu.async_copy(src_hbm.at[pl.ds(base0, _EPW)], sidx_all, isem)

    def zrow(i, _):
        for j in range(_D // 16):
            r0[i, pl.ds(j * 16, 16)] = jnp.zeros((16,), jnp.float32)
        return 0

    lax.fori_loop(0, _CHS, zrow, 0)

    rbase = sid * _RA
    for k in range(4):
        pltpu.sync_copy(r0, acc.at[pl.ds(rbase + k * _CHS, _CHS)])
    pltpu.sync_copy(r0.at[pl.ds(0, _RA - 4 * _CHS)],
                    acc.at[pl.ds(rbase + 4 * _CHS, _RA - 4 * _CHS)])

    @pl.when(sid == _NS - 1)
    def _():
        pltpu.sync_copy(r0.at[pl.ds(0, _RLAST_EXTRA)],
                        acc.at[pl.ds(_NS * _RA, _RLAST_EXTRA)])

    isd.wait()
    plsc.subcore_barrier()

    def rnd(r, _):
        gdescs = []
        ddescs = []
        for b in range(_NB4):
            i = r * _NB4 + b

            @pl.when(r > 0)
            def _():
                pltpu.make_async_copy(rows[b], acc.at[dcur[b]],
                                      ssem[b]).wait()

            ddescs.append(pltpu.async_copy(
                dst_hbm.at[pl.ds(base0 + i * _CHS, _CHS)], dcur[b], dsem[b]))
            gdescs.append(pltpu.async_copy(
                u_hbm.at[sidx_all.at[pl.ds(i * _CHS, _CHS)]], rows[b],
                gsem[b]))
        for b in range(_NB4):
            ddescs[b].wait()
            gdescs[b].wait()
            pltpu.async_copy(rows[b], acc.at[dcur[b]], ssem[b], add=True)
        return 0

    lax.fori_loop(0, _NRS, rnd, 0)
    for b in range(_NB4):
        pltpu.make_async_copy(rows[b], acc.at[dcur[b]], ssem[b]).wait()

    sidx16[pl.ds(0, _REMS)] = sidx_all[pl.ds(_NFULLS * _CHS, _REMS)]
    pltpu.async_copy(dst_hbm.at[pl.ds(base0 + _NFULLS * _CHS, _REMS)],
                     didx16, ds0).wait()
    pltpu.async_copy(u_hbm.at[sidx16], rows16, gs0).wait()
    pltpu.sync_copy(rows16, acc.at[didx16], add=True)

    plsc.subcore_barrier()

    for k in range(4):
        pltpu.sync_copy(acc.at[pl.ds(rbase + k * _CHS, _CHS)],
                        out_hbm.at[cid, pl.ds(rbase + k * _CHS, _CHS)])
    pltpu.sync_copy(acc.at[pl.ds(rbase + 4 * _CHS, _RA - 4 * _CHS)],
                    out_hbm.at[cid, pl.ds(rbase + 4 * _CHS, _RA - 4 * _CHS)])

    @pl.when(sid == _NS - 1)
    def _():
        pltpu.sync_copy(acc.at[pl.ds(_NS * _RA, _RLAST_EXTRA)],
                        out_hbm.at[cid, pl.ds(_NS * _RA, _RLAST_EXTRA)])


_scat_call = pl.kernel(
    _scat_body,
    out_type=jax.ShapeDtypeStruct((_NC, _N, _D), jnp.float32),
    mesh=_mesh,
    scratch_types=[
        pltpu.VMEM_SHARED((_N, _D), jnp.float32),
        pltpu.VMEM((_EPW,), jnp.int32),
        pltpu.VMEM((_CHS, _D), jnp.float32),
        pltpu.VMEM((_CHS, _D), jnp.float32),
        pltpu.VMEM((_CHS,), jnp.int32),
        pltpu.VMEM((_CHS,), jnp.int32),
        pltpu.VMEM((_REMS, _D), jnp.float32),
        pltpu.VMEM((_REMS,), jnp.int32),
        pltpu.VMEM((_REMS,), jnp.int32),
        pltpu.SemaphoreType.DMA,
        pltpu.SemaphoreType.DMA,
        pltpu.SemaphoreType.DMA,
        pltpu.SemaphoreType.DMA,
        pltpu.SemaphoreType.DMA,
        pltpu.SemaphoreType.DMA,
        pltpu.SemaphoreType.DMA,
    ],
)


def _tc1_body(x_ref, w1_ref, degp_ref, u1_ref, dis_ref):
    deg = degp_ref[0] + degp_ref[1] + 1.0
    dis = lax.rsqrt(deg)
    h = jnp.dot(x_ref[...], w1_ref[...], preferred_element_type=jnp.float32)
    u1_ref[...] = h * dis
    dis_ref[...] = dis


_tc1 = pl.pallas_call(
    _tc1_body,
    out_shape=(
        jax.ShapeDtypeStruct((_N, _D), jnp.float32),
        jax.ShapeDtypeStruct((_N, 1), jnp.float32),
    ),
)


def _tc2_body(scatp_ref, u1_ref, dis_ref, w2_ref, b1_ref, u2_ref):
    s = scatp_ref[0] + scatp_ref[1] + u1_ref[...]
    z1 = jnp.maximum(dis_ref[...] * s + b1_ref[...], 0.0)
    u2_ref[...] = jnp.dot(z1, w2_ref[...],
                          preferred_element_type=jnp.float32) * dis_ref[...]


_tc2 = pl.pallas_call(
    _tc2_body,
    out_shape=jax.ShapeDtypeStruct((_N, _D), jnp.float32),
)


def _tc3_body(scatp_ref, u2_ref, dis_ref, b2_ref, batch_ref, wc_ref, bc_ref,
              out_ref):
    s = scatp_ref[0] + scatp_ref[1] + u2_ref[...]
    z2 = jnp.maximum(dis_ref[...] * s + b2_ref[...], 0.0)
    bvec = batch_ref[...]
    gids = lax.broadcasted_iota(jnp.int32, (_G, _N), 0)
    pmat = (gids == bvec).astype(jnp.float32)
    cnt = jnp.sum(pmat, axis=1, keepdims=True)
    g = jnp.dot(pmat, z2, preferred_element_type=jnp.float32)
    g = g / jnp.maximum(cnt, 1.0)
    logits = jnp.dot(g, wc_ref[...],
                     preferred_element_type=jnp.float32) + bc_ref[...]
    m = jnp.max(logits, axis=1, keepdims=True)
    e = logits - m
    out_ref[...] = e - jnp.log(jnp.sum(jnp.exp(e), axis=1, keepdims=True))


_tc3 = pl.pallas_call(
    _tc3_body,
    out_shape=jax.ShapeDtypeStruct((_G, _C), jnp.float32),
)


def kernel(x, edge_index, batch, W1, b1, W2, b2, Wc, bc):
    src = edge_index[0].astype(jnp.int32)
    dst = edge_index[1].astype(jnp.int32)
    batch2 = batch.astype(jnp.int32).reshape(1, _N)
    degp = _deg_call(dst)
    degp3 = degp.reshape(_NC, _N, 1)
    u1, dis = _tc1(x, W1, degp3)
    scat1 = _scat_call(u1, src, dst)
    u2 = _tc2(scat1, u1, dis, W2, b1.reshape(1, _D))
    scat2 = _scat_call(u2, src, dst)
    return _tc3(scat2, u2, dis, b2.reshape(1, _D), batch2, Wc,
                bc.reshape(1, _C))


# speedup vs baseline: 1.1487x; 1.1487x over previous
import jax
import jax.numpy as jnp
from jax import lax
from jax.experimental import pallas as pl
from jax.experimental.pallas import tpu as pltpu
from jax.experimental.pallas import tpu_sc as plsc

_N = 10000
_E = 320000
_D = 128
_G = 256
_C = 60

_NC = 2
_NS = 16
_NW = _NC * _NS
_EPW = _E // _NW
_CH = 128
_NFULL = _EPW // _CH
_REM = _EPW - _NFULL * _CH
_NB = 3
_NR = _NFULL // _NB

_CHS = 32
_NFULLS = _EPW // _CHS
_REMS = _EPW - _NFULLS * _CHS
_NB4 = 8
_NRS = _NFULLS // _NB4

_RA = 624
_RLAST_EXTRA = _N - 16 * _RA

_mesh = plsc.VectorSubcoreMesh(
    core_axis_name="c", subcore_axis_name="s", num_cores=_NC, num_subcores=_NS
)


def _copy_idx_regs(dst_ref, src_ref, base, n):
    for j in range(n // 16):
        dst_ref[pl.ds(j * 16, 16)] = src_ref[pl.ds(base + j * 16, 16)]


def _deg_body(dst_hbm, out_hbm, acc, didx_all, zval, ones, idx16, ones16,
              dc0, dc1, dc2, isem, ss0, ss1, ss2):
    cid = lax.axis_index("c")
    sid = lax.axis_index("s")
    wid = sid * _NC + cid
    base0 = wid * _EPW
    dcur = (dc0, dc1, dc2)
    ssem = (ss0, ss1, ss2)

    idesc = pltpu.async_copy(dst_hbm.at[pl.ds(base0, _EPW)], didx_all, isem)

    def fill_z(i, _):
        zval[pl.ds(i * 16, 16)] = jnp.zeros((16,), jnp.float32)
        return 0

    lax.fori_loop(0, 640 // 16, fill_z, 0)

    def fill_o(i, _):
        ones[pl.ds(i * 16, 16)] = jnp.ones((16,), jnp.float32)
        return 0

    lax.fori_loop(0, _CH // 16, fill_o, 0)
    ones16[pl.ds(0, 16)] = jnp.ones((16,), jnp.float32)

    @pl.when(sid < _NS - 1)
    def _():
        pltpu.sync_copy(zval.at[pl.ds(0, _RA)], acc.at[pl.ds(sid * _RA, _RA)])

    @pl.when(sid == _NS - 1)
    def _():
        pltpu.sync_copy(zval, acc.at[pl.ds((_NS - 1) * _RA, _RA + _RLAST_EXTRA)])

    idesc.wait()
    plsc.subcore_barrier()

    def rnd(r, _):
        for b in range(_NB):
            i = r * _NB + b

            @pl.when(r > 0)
            def _():
                pltpu.make_async_copy(ones, acc.at[dcur[b]], ssem[b]).wait()

            _copy_idx_regs(dcur[b], didx_all, i * _CH, _CH)
            pltpu.async_copy(ones, acc.at[dcur[b]], ssem[b], add=True)
        return 0

    lax.fori_loop(0, _NR, rnd, 0)
    for b in range(_NB):
        pltpu.make_async_copy(ones, acc.at[dcur[b]], ssem[b]).wait()

    idx16[pl.ds(0, _REM)] = didx_all[pl.ds(_NFULL * _CH, _REM)]
    pltpu.sync_copy(ones16, acc.at[idx16], add=True)

    plsc.subcore_barrier()

    @pl.when(sid == 0)
    def _():
        pltpu.sync_copy(acc, out_hbm.at[cid])


_deg_call = pl.kernel(
    _deg_body,
    out_type=jax.ShapeDtypeStruct((_NC, _N), jnp.float32),
    mesh=_mesh,
    scratch_types=[
        pltpu.VMEM_SHARED((_N,), jnp.float32),
        pltpu.VMEM((_EPW,), jnp.int32),
        pltpu.VMEM((640,), jnp.float32),
        pltpu.VMEM((_CH,), jnp.float32),
        pltpu.VMEM((_REM,), jnp.int32),
        pltpu.VMEM((_REM,), jnp.float32),
        pltpu.VMEM((_CH,), jnp.int32),
        pltpu.VMEM((_CH,), jnp.int32),
        pltpu.VMEM((_CH,), jnp.int32),
        pltpu.SemaphoreType.DMA,
        pltpu.SemaphoreType.DMA,
        pltpu.SemaphoreType.DMA,
        pltpu.SemaphoreType.DMA,
    ],
)


def _scat_body(u_hbm, src_hbm, dst_hbm, out_hbm, acc, sidx_all,
               r0, r1, r2, r3, r4, r5, r6, r7,
               dc0, dc1, dc2, dc3, dc4, dc5, dc6, dc7,
               rows16, sidx16, didx16,
               isem, gs0, gs1, gs2, gs3, gs4, gs5, gs6, gs7,
               ss0, ss1, ss2, ss3, ss4, ss5, ss6, ss7,
               ds0, ds1, ds2, ds3, ds4, ds5, ds6, ds7):
    cid = lax.axis_index("c")
    sid = lax.axis_index("s")
    wid = sid * _NC + cid
    base0 = wid * _EPW
    rows = (r0, r1, r2, r3, r4, r5, r6, r7)
    dcur = (dc0, dc1, dc2, dc3, dc4, dc5, dc6, dc7)
    gsem = (gs0, gs1, gs2, gs3, gs4, gs5, gs6, gs7)
    ssem = (ss0, ss1, ss2, ss3, ss4, ss5, ss6, ss7)
    dsem = (ds0, ds1, ds2, ds3, ds4, ds5, ds6, ds7)

    isd = pltpu.async_copy(src_hbm.at[pl.ds(base0, _EPW)], sidx_all, isem)

    def zrow(i, _):
        for j in range(_D // 16):
            r0[i, pl.ds(j * 16, 16)] = jnp.zeros((16,), jnp.float32)
        return 0

    lax.fori_loop(0, _CHS, zrow, 0)

    rbase = sid * _RA
    for k in range(19):
        pltpu.sync_copy(r0, acc.at[pl.ds(rbase + k * _CHS, _CHS)])
    pltpu.sync_copy(r0.at[pl.ds(0, _RA - 19 * _CHS)],
                    acc.at[pl.ds(rbase + 19 * _CHS, _RA - 19 * _CHS)])

    @pl.when(sid == _NS - 1)
    def _():
        pltpu.sync_copy(r0.at[pl.ds(0, _RLAST_EXTRA)],
                        acc.at[pl.ds(_NS * _RA, _RLAST_EXTRA)])

    isd.wait()
    plsc.subcore_barrier()

    def rnd(r, _):
        gdescs = []
        ddescs = []
        for b in range(_NB4):
            i = r * _NB4 + b

            @pl.when(r > 0)
            def _():
                pltpu.make_async_copy(rows[b], acc.at[dcur[b]],
                                      ssem[b]).wait()

            ddescs.append(pltpu.async_copy(
                dst_hbm.at[pl.ds(base0 + i * _CHS, _CHS)], dcur[b], dsem[b]))
            gdescs.append(pltpu.async_copy(
                u_hbm.at[sidx_all.at[pl.ds(i * _CHS, _CHS)]], rows[b],
                gsem[b]))
        for b in range(_NB4):
            ddescs[b].wait()
            gdescs[b].wait()
            pltpu.async_copy(rows[b], acc.at[dcur[b]], ssem[b], add=True)
        return 0

    lax.fori_loop(0, _NRS, rnd, 0)
    for b in range(_NB4):
        pltpu.make_async_copy(rows[b], acc.at[dcur[b]], ssem[b]).wait()

    sidx16[pl.ds(0, _REMS)] = sidx_all[pl.ds(_NFULLS * _CHS, _REMS)]
    pltpu.async_copy(dst_hbm.at[pl.ds(base0 + _NFULLS * _CHS, _REMS)],
                     didx16, ds0).wait()
    pltpu.async_copy(u_hbm.at[sidx16], rows16, gs0).wait()
    pltpu.sync_copy(rows16, acc.at[didx16], add=True)

    plsc.subcore_barrier()

    for k in range(19):
        pltpu.sync_copy(acc.at[pl.ds(rbase + k * _CHS, _CHS)],
                        out_hbm.at[cid, pl.ds(rbase + k * _CHS, _CHS)])
    pltpu.sync_copy(acc.at[pl.ds(rbase + 19 * _CHS, _RA - 19 * _CHS)],
                    out_hbm.at[cid, pl.ds(rbase + 19 * _CHS, _RA - 19 * _CHS)])

    @pl.when(sid == _NS - 1)
    def _():
        pltpu.sync_copy(acc.at[pl.ds(_NS * _RA, _RLAST_EXTRA)],
                        out_hbm.at[cid, pl.ds(_NS * _RA, _RLAST_EXTRA)])


_scat_call = pl.kernel(
    _scat_body,
    out_type=jax.ShapeDtypeStruct((_NC, _N, _D), jnp.float32),
    mesh=_mesh,
    scratch_types=[
        pltpu.VMEM_SHARED((_N, _D), jnp.float32),
        pltpu.VMEM((_EPW,), jnp.int32),
        pltpu.VMEM((_CHS, _D), jnp.float32),
        pltpu.VMEM((_CHS, _D), jnp.float32),
        pltpu.VMEM((_CHS, _D), jnp.float32),
        pltpu.VMEM((_CHS, _D), jnp.float32),
        pltpu.VMEM((_CHS, _D), jnp.float32),
        pltpu.VMEM((_CHS, _D), jnp.float32),
        pltpu.VMEM((_CHS, _D), jnp.float32),
        pltpu.VMEM((_CHS, _D), jnp.float32),
        pltpu.VMEM((_CHS,), jnp.int32),
        pltpu.VMEM((_CHS,), jnp.int32),
        pltpu.VMEM((_CHS,), jnp.int32),
        pltpu.VMEM((_CHS,), jnp.int32),
        pltpu.VMEM((_CHS,), jnp.int32),
        pltpu.VMEM((_CHS,), jnp.int32),
        pltpu.VMEM((_CHS,), jnp.int32),
        pltpu.VMEM((_CHS,), jnp.int32),
        pltpu.VMEM((_REMS, _D), jnp.float32),
        pltpu.VMEM((_REMS,), jnp.int32),
        pltpu.VMEM((_REMS,), jnp.int32),
        pltpu.SemaphoreType.DMA,
        pltpu.SemaphoreType.DMA,
        pltpu.SemaphoreType.DMA,
        pltpu.SemaphoreType.DMA,
        pltpu.SemaphoreType.DMA,
        pltpu.SemaphoreType.DMA,
        pltpu.SemaphoreType.DMA,
        pltpu.SemaphoreType.DMA,
        pltpu.SemaphoreType.DMA,
        pltpu.SemaphoreType.DMA,
        pltpu.SemaphoreType.DMA,
        pltpu.SemaphoreType.DMA,
        pltpu.SemaphoreType.DMA,
        pltpu.SemaphoreType.DMA,
        pltpu.SemaphoreType.DMA,
        pltpu.SemaphoreType.DMA,
        pltpu.SemaphoreType.DMA,
        pltpu.SemaphoreType.DMA,
        pltpu.SemaphoreType.DMA,
        pltpu.SemaphoreType.DMA,
        pltpu.SemaphoreType.DMA,
        pltpu.SemaphoreType.DMA,
        pltpu.SemaphoreType.DMA,
        pltpu.SemaphoreType.DMA,
        pltpu.SemaphoreType.DMA,
    ],
)


def _tc1_body(x_ref, w1_ref, degp_ref, u1_ref, dis_ref):
    deg = degp_ref[0] + degp_ref[1] + 1.0
    dis = lax.rsqrt(deg)
    h = jnp.dot(x_ref[...], w1_ref[...], preferred_element_type=jnp.float32)
    u1_ref[...] = h * dis
    dis_ref[...] = dis


_tc1 = pl.pallas_call(
    _tc1_body,
    out_shape=(
        jax.ShapeDtypeStruct((_N, _D), jnp.float32),
        jax.ShapeDtypeStruct((_N, 1), jnp.float32),
    ),
)


def _tc2_body(scatp_ref, u1_ref, dis_ref, w2_ref, b1_ref, u2_ref):
    s = scatp_ref[0] + scatp_ref[1] + u1_ref[...]
    z1 = jnp.maximum(dis_ref[...] * s + b1_ref[...], 0.0)
    u2_ref[...] = jnp.dot(z1, w2_ref[...],
                          preferred_element_type=jnp.float32) * dis_ref[...]


_tc2 = pl.pallas_call(
    _tc2_body,
    out_shape=jax.ShapeDtypeStruct((_N, _D), jnp.float32),
)


def _tc3_body(scatp_ref, u2_ref, dis_ref, b2_ref, batch_ref, wc_ref, bc_ref,
              out_ref):
    s = scatp_ref[0] + scatp_ref[1] + u2_ref[...]
    z2 = jnp.maximum(dis_ref[...] * s + b2_ref[...], 0.0)
    bvec = batch_ref[...]
    gids = lax.broadcasted_iota(jnp.int32, (_G, _N), 0)
    pmat = (gids == bvec).astype(jnp.float32)
    cnt = jnp.sum(pmat, axis=1, keepdims=True)
    g = jnp.dot(pmat, z2, preferred_element_type=jnp.float32)
    g = g / jnp.maximum(cnt, 1.0)
    logits = jnp.dot(g, wc_ref[...],
                     preferred_element_type=jnp.float32) + bc_ref[...]
    m = jnp.max(logits, axis=1, keepdims=True)
    e = logits - m
    out_ref[...] = e - jnp.log(jnp.sum(jnp.exp(e), axis=1, keepdims=True))


_tc3 = pl.pallas_call(
    _tc3_body,
    out_shape=jax.ShapeDtypeStruct((_G, _C), jnp.float32),
)


def kernel(x, edge_index, batch, W1, b1, W2, b2, Wc, bc):
    src = edge_index[0].astype(jnp.int32)
    dst = edge_index[1].astype(jnp.int32)
    batch2 = batch.astype(jnp.int32).reshape(1, _N)
    degp = _deg_call(dst)
    degp3 = degp.reshape(_NC, _N, 1)
    u1, dis = _tc1(x, W1, degp3)
    scat1 = _scat_call(u1, src, dst)
    u2 = _tc2(scat1, u1, dis, W2, b1.reshape(1, _D))
    scat2 = _scat_call(u2, src, dst)
    return _tc3(scat2, u2, dis, b2.reshape(1, _D), batch2, Wc,
                bc.reshape(1, _C))


# speedup vs baseline: 1.1943x; 1.0398x over previous
import jax
import jax.numpy as jnp
from jax import lax
from jax.experimental import pallas as pl
from jax.experimental.pallas import tpu as pltpu
from jax.experimental.pallas import tpu_sc as plsc

_N = 10000
_E = 320000
_D = 128
_G = 256
_C = 60

_NC = 2
_NS = 16
_NW = _NC * _NS
_EPW = _E // _NW
_CH = 128
_NFULL = _EPW // _CH
_REM = _EPW - _NFULL * _CH
_NB = 3
_NR = _NFULL // _NB

_CHS = 64
_NFULLS = _EPW // _CHS
_REMS = _EPW - _NFULLS * _CHS
_NB4 = 4
_NRS = _NFULLS // _NB4

_RA = 624
_RLAST_EXTRA = _N - 16 * _RA

_mesh = plsc.VectorSubcoreMesh(
    core_axis_name="c", subcore_axis_name="s", num_cores=_NC, num_subcores=_NS
)


def _copy_idx_regs(dst_ref, src_ref, base, n):
    for j in range(n // 16):
        dst_ref[pl.ds(j * 16, 16)] = src_ref[pl.ds(base + j * 16, 16)]


def _deg_body(dst_hbm, out_hbm, acc, didx_all, zval, ones, idx16, ones16,
              dc0, dc1, dc2, isem, ss0, ss1, ss2):
    cid = lax.axis_index("c")
    sid = lax.axis_index("s")
    wid = sid * _NC + cid
    base0 = wid * _EPW
    dcur = (dc0, dc1, dc2)
    ssem = (ss0, ss1, ss2)

    idesc = pltpu.async_copy(dst_hbm.at[pl.ds(base0, _EPW)], didx_all, isem)

    def fill_z(i, _):
        zval[pl.ds(i * 16, 16)] = jnp.zeros((16,), jnp.float32)
        return 0

    lax.fori_loop(0, 640 // 16, fill_z, 0)

    def fill_o(i, _):
        ones[pl.ds(i * 16, 16)] = jnp.ones((16,), jnp.float32)
        return 0

    lax.fori_loop(0, _CH // 16, fill_o, 0)
    ones16[pl.ds(0, 16)] = jnp.ones((16,), jnp.float32)

    @pl.when(sid < _NS - 1)
    def _():
        pltpu.sync_copy(zval.at[pl.ds(0, _RA)], acc.at[pl.ds(sid * _RA, _RA)])

    @pl.when(sid == _NS - 1)
    def _():
        pltpu.sync_copy(zval, acc.at[pl.ds((_NS - 1) * _RA, _RA + _RLAST_EXTRA)])

    idesc.wait()
    plsc.subcore_barrier()

    def rnd(r, _):
        for b in range(_NB):
            i = r * _NB + b

            @pl.when(r > 0)
            def _():
                pltpu.make_async_copy(ones, acc.at[dcur[b]], ssem[b]).wait()

            _copy_idx_regs(dcur[b], didx_all, i * _CH, _CH)
            pltpu.async_copy(ones, acc.at[dcur[b]], ssem[b], add=True)
        return 0

    lax.fori_loop(0, _NR, rnd, 0)
    for b in range(_NB):
        pltpu.make_async_copy(ones, acc.at[dcur[b]], ssem[b]).wait()

    idx16[pl.ds(0, _REM)] = didx_all[pl.ds(_NFULL * _CH, _REM)]
    pltpu.sync_copy(ones16, acc.at[idx16], add=True)

    plsc.subcore_barrier()

    @pl.when(sid == 0)
    def _():
        pltpu.sync_copy(acc, out_hbm.at[cid])


_deg_call = pl.kernel(
    _deg_body,
    out_type=jax.ShapeDtypeStruct((_NC, _N), jnp.float32),
    mesh=_mesh,
    scratch_types=[
        pltpu.VMEM_SHARED((_N,), jnp.float32),
        pltpu.VMEM((_EPW,), jnp.int32),
        pltpu.VMEM((640,), jnp.float32),
        pltpu.VMEM((_CH,), jnp.float32),
        pltpu.VMEM((_REM,), jnp.int32),
        pltpu.VMEM((_REM,), jnp.float32),
        pltpu.VMEM((_CH,), jnp.int32),
        pltpu.VMEM((_CH,), jnp.int32),
        pltpu.VMEM((_CH,), jnp.int32),
        pltpu.SemaphoreType.DMA,
        pltpu.SemaphoreType.DMA,
        pltpu.SemaphoreType.DMA,
        pltpu.SemaphoreType.DMA,
    ],
)


def _scat_body(u_hbm, src_hbm, dst_hbm, out_hbm, acc, sidx_all,
               r0, r1, r2, r3, dc0, dc1, dc2, dc3, rows16, sidx16, didx16,
               isem, gs0, gs1, gs2, gs3, ss0, ss1, ss2, ss3,
               ds0, ds1, ds2, ds3):
    cid = lax.axis_index("c")
    sid = lax.axis_index("s")
    wid = sid * _NC + cid
    base0 = wid * _EPW
    rows = (r0, r1, r2, r3)
    dcur = (dc0, dc1, dc2, dc3)
    gsem = (gs0, gs1, gs2, gs3)
    ssem = (ss0, ss1, ss2, ss3)
    dsem = (ds0, ds1, ds2, ds3)

    isd = pltpu.async_copy(src_hbm.at[pl.ds(base0, _EPW)], sidx_all, isem)

    def zrow(i, _):
        for j in range(_D // 16):
            r0[i, pl.ds(j * 16, 16)] = jnp.zeros((16,), jnp.float32)
        return 0

    lax.fori_loop(0, _CHS, zrow, 0)

    rbase = sid * _RA
    for k in range(9):
        pltpu.sync_copy(r0, acc.at[pl.ds(rbase + k * _CHS, _CHS)])
    pltpu.sync_copy(r0.at[pl.ds(0, _RA - 9 * _CHS)],
                    acc.at[pl.ds(rbase + 9 * _CHS, _RA - 9 * _CHS)])

    @pl.when(sid == _NS - 1)
    def _():
        pltpu.sync_copy(r0.at[pl.ds(0, _RLAST_EXTRA)],
                        acc.at[pl.ds(_NS * _RA, _RLAST_EXTRA)])

    isd.wait()
    plsc.subcore_barrier()

    def rnd(r, _):
        gdescs = []
        ddescs = []
        for b in range(_NB4):
            i = r * _NB4 + b

            @pl.when(r > 0)
            def _():
                pltpu.make_async_copy(rows[b], acc.at[dcur[b]],
                                      ssem[b]).wait()

            ddescs.append(pltpu.async_copy(
                dst_hbm.at[pl.ds(base0 + i * _CHS, _CHS)], dcur[b], dsem[b]))
            gdescs.append(pltpu.async_copy(
                u_hbm.at[sidx_all.at[pl.ds(i * _CHS, _CHS)]], rows[b],
                gsem[b]))
        for b in range(_NB4):
            ddescs[b].wait()
            gdescs[b].wait()
            pltpu.async_copy(rows[b], acc.at[dcur[b]], ssem[b], add=True)
        return 0

    lax.fori_loop(0, _NRS, rnd, 0)
    for b in range(_NB4):
        pltpu.make_async_copy(rows[b], acc.at[dcur[b]], ssem[b]).wait()

    sidx16[pl.ds(0, _REMS)] = sidx_all[pl.ds(_NFULLS * _CHS, _REMS)]
    pltpu.async_copy(dst_hbm.at[pl.ds(base0 + _NFULLS * _CHS, _REMS)],
                     didx16, ds0).wait()
    pltpu.async_copy(u_hbm.at[sidx16], rows16, gs0).wait()
    pltpu.sync_copy(rows16, acc.at[didx16], add=True)

    plsc.subcore_barrier()

    for k in range(9):
        pltpu.sync_copy(acc.at[pl.ds(rbase + k * _CHS, _CHS)],
                        out_hbm.at[cid, pl.ds(rbase + k * _CHS, _CHS)])
    pltpu.sync_copy(acc.at[pl.ds(rbase + 9 * _CHS, _RA - 9 * _CHS)],
                    out_hbm.at[cid, pl.ds(rbase + 9 * _CHS, _RA - 9 * _CHS)])

    @pl.when(sid == _NS - 1)
    def _():
        pltpu.sync_copy(acc.at[pl.ds(_NS * _RA, _RLAST_EXTRA)],
                        out_hbm.at[cid, pl.ds(_NS * _RA, _RLAST_EXTRA)])


_scat_call = pl.kernel(
    _scat_body,
    out_type=jax.ShapeDtypeStruct((_NC, _N, _D), jnp.float32),
    mesh=_mesh,
    scratch_types=[
        pltpu.VMEM_SHARED((_N, _D), jnp.float32),
        pltpu.VMEM((_EPW,), jnp.int32),
        pltpu.VMEM((_CHS, _D), jnp.float32),
        pltpu.VMEM((_CHS, _D), jnp.float32),
        pltpu.VMEM((_CHS, _D), jnp.float32),
        pltpu.VMEM((_CHS, _D), jnp.float32),
        pltpu.VMEM((_CHS,), jnp.int32),
        pltpu.VMEM((_CHS,), jnp.int32),
        pltpu.VMEM((_CHS,), jnp.int32),
        pltpu.VMEM((_CHS,), jnp.int32),
        pltpu.VMEM((_REMS, _D), jnp.float32),
        pltpu.VMEM((_REMS,), jnp.int32),
        pltpu.VMEM((_REMS,), jnp.int32),
        pltpu.SemaphoreType.DMA,
        pltpu.SemaphoreType.DMA,
        pltpu.SemaphoreType.DMA,
        pltpu.SemaphoreType.DMA,
        pltpu.SemaphoreType.DMA,
        pltpu.SemaphoreType.DMA,
        pltpu.SemaphoreType.DMA,
        pltpu.SemaphoreType.DMA,
        pltpu.SemaphoreType.DMA,
        pltpu.SemaphoreType.DMA,
        pltpu.SemaphoreType.DMA,
        pltpu.SemaphoreType.DMA,
        pltpu.SemaphoreType.DMA,
    ],
)


def _tc1_body(x_ref, w1_ref, degp_ref, u1_ref, dis_ref):
    deg = degp_ref[0] + degp_ref[1] + 1.0
    dis = lax.rsqrt(deg)
    h = jnp.dot(x_ref[...], w1_ref[...], preferred_element_type=jnp.float32)
    u1_ref[...] = h * dis
    dis_ref[...] = dis


_tc1 = pl.pallas_call(
    _tc1_body,
    out_shape=(
        jax.ShapeDtypeStruct((_N, _D), jnp.float32),
        jax.ShapeDtypeStruct((_N, 1), jnp.float32),
    ),
)


def _tc2_body(scatp_ref, u1_ref, dis_ref, w2_ref, b1_ref, u2_ref):
    s = scatp_ref[0] + scatp_ref[1] + u1_ref[...]
    z1 = jnp.maximum(dis_ref[...] * s + b1_ref[...], 0.0)
    u2_ref[...] = jnp.dot(z1, w2_ref[...],
                          preferred_element_type=jnp.float32) * dis_ref[...]


_tc2 = pl.pallas_call(
    _tc2_body,
    out_shape=jax.ShapeDtypeStruct((_N, _D), jnp.float32),
)


def _tc3_body(scatp_ref, u2_ref, dis_ref, b2_ref, batch_ref, wc_ref, bc_ref,
              out_ref):
    s = scatp_ref[0] + scatp_ref[1] + u2_ref[...]
    z2 = jnp.maximum(dis_ref[...] * s + b2_ref[...], 0.0)
    bvec = batch_ref[...]
    gids = lax.broadcasted_iota(jnp.int32, (_G, _N), 0)
    pmat = (gids == bvec).astype(jnp.float32)
    cnt = jnp.sum(pmat, axis=1, keepdims=True)
    g = jnp.dot(pmat, z2, preferred_element_type=jnp.float32)
    g = g / jnp.maximum(cnt, 1.0)
    logits = jnp.dot(g, wc_ref[...],
                     preferred_element_type=jnp.float32) + bc_ref[...]
    m = jnp.max(logits, axis=1, keepdims=True)
    e = logits - m
    out_ref[...] = e - jnp.log(jnp.sum(jnp.exp(e), axis=1, keepdims=True))


_tc3 = pl.pallas_call(
    _tc3_body,
    out_shape=jax.ShapeDtypeStruct((_G, _C), jnp.float32),
)


def kernel(x, edge_index, batch, W1, b1, W2, b2, Wc, bc):
    src = edge_index[0].astype(jnp.int32)
    dst = edge_index[1].astype(jnp.int32)
    batch2 = batch.astype(jnp.int32).reshape(1, _N)
    degp = _deg_call(dst)
    degp3 = degp.reshape(_NC, _N, 1)
    u1, dis = _tc1(x, W1, degp3)
    scat1 = _scat_call(u1, src, dst)
    u2 = _tc2(scat1, u1, dis, W2, b1.reshape(1, _D))
    scat2 = _scat_call(u2, src, dst)
    return _tc3(scat2, u2, dis, b2.reshape(1, _D), batch2, Wc,
                bc.reshape(1, _C))
